# baseline (speedup 1.0000x reference)
import jax
import jax.numpy as jnp
import numpy as np
from jax.experimental import pallas as pl

E = 64
H = 64
NH = 8
HD = H // NH
T2 = 2048


def _gelu(x):
    return 0.5 * x * (1.0 + jax.lax.erf(x * np.float32(1.0 / np.sqrt(2.0))))


def _router_kernel(cfa_ref, cfb_ref, hfa_ref, hfb_ref,
                   wc_ref, bc_ref, gc_ref, bec_ref,
                   wh_ref, bh_ref, gh_ref, beh_ref,
                   wqkv_ref, bqkv_ref,
                   wo_ref, bo_ref,
                   wf_ref, bf_ref, gf_ref, bef_ref,
                   whead_ref, bhead_ref,
                   w2rb_ref, b2rb_ref, w2un_ref, b2un_ref,
                   m1_ref, sels_ref, selt_ref,
                   rb_ref, unc_ref):
    f32 = jnp.float32
    mm = lambda a, b: jnp.dot(a, b, preferred_element_type=f32)

    m1 = m1_ref[...]
    sels = sels_ref[...]
    selt = selt_ref[...]

    def segln(x, g, b):
        m = mm(x, m1)
        c = x - m
        v = mm(c * c, m1)
        return c * jax.lax.rsqrt(v + 1e-5) * g + b

    wc = wc_ref[...]
    pre_c = jnp.concatenate([mm(cfa_ref[...], wc), mm(cfb_ref[...], wc)],
                            axis=1) + bc_ref[...]
    ce = _gelu(segln(pre_c, gc_ref[...], bec_ref[...]))
    wh = wh_ref[...]
    pre_h = jnp.concatenate([mm(hfa_ref[...], wh), mm(hfb_ref[...], wh)],
                            axis=1) + bh_ref[...]
    he = _gelu(segln(pre_h, gh_ref[...], beh_ref[...]))

    bqkv = bqkv_ref[...]
    bq = bqkv[:, 0:128]; bk = bqkv[:, 128:256]; bv = bqkv[:, 256:384]
    qkv_c = mm(ce, wqkv_ref[...]) + bqkv
    qkv_h = mm(he, wqkv_ref[...]) + bqkv
    q_c = qkv_c[:, 0:128]; k_c = qkv_c[:, 128:256]; v_c = qkv_c[:, 256:384]
    q_h = qkv_h[:, 0:128]; k_h = qkv_h[:, 128:256]; v_h = qkv_h[:, 256:384]

    s_cc = mm(q_c * k_c, sels)
    s_ch = mm(q_c * k_h, sels)
    s_ct = mm(q_c * bk, sels)
    s_hc = mm(q_h * k_c, sels)
    s_hh = mm(q_h * k_h, sels)
    s_ht = mm(q_h * bk, sels)
    s_tc = mm(bq * k_c, sels)
    s_th = mm(bq * k_h, sels)
    s_tt = mm(bq * bk, sels)

    def softmax3(a, b, c):
        m = jnp.maximum(jnp.maximum(a, b), c)
        ea = jnp.exp(a - m); eb = jnp.exp(b - m); ec = jnp.exp(c - m)
        inv = 1.0 / (ea + eb + ec)
        return ea * inv, eb * inv, ec * inv

    a_cc, a_ch, a_ct = softmax3(s_cc, s_ch, s_ct)
    a_hc, a_hh, a_ht = softmax3(s_hc, s_hh, s_ht)
    a_tc, a_th, a_tt = softmax3(s_tc, s_th, jnp.zeros_like(s_tc) + s_tt)

    third = np.float32(1.0 / 3.0)
    w_vc = (a_cc + a_hc + a_tc) * third
    w_vh = (a_ch + a_hh + a_th) * third
    w_vt = (a_ct + a_ht + a_tt) * third

    o = (mm(w_vc, selt) * v_c + mm(w_vh, selt) * v_h + mm(w_vt, selt) * bv)
    att_mean = mm(o, wo_ref[...]) + bo_ref[...]

    fused = _gelu(segln(mm(att_mean, wf_ref[...]) + bf_ref[...],
                        gf_ref[...], bef_ref[...]))

    hh = _gelu(mm(fused, whead_ref[...]) + bhead_ref[...])
    h1 = hh[:, 0:64]
    hu = hh[:, 64:96]
    rb = jnp.tanh(mm(h1, w2rb_ref[...]) + b2rb_ref[...])
    unc = jnp.logaddexp(mm(hu, w2un_ref[...]) + b2un_ref[...], 0.0)

    rb_ref[0:T2, :] = rb[:, 0:E]
    rb_ref[T2:2 * T2, :] = rb[:, E:2 * E]
    unc_ref[0:T2, :] = unc[:, 0:E]
    unc_ref[T2:2 * T2, :] = unc[:, E:2 * E]


def _bd(a, b):
    (r1, c1), (r2, c2) = a.shape, b.shape
    z = jnp.zeros((r1 + r2, c1 + c2), jnp.float32)
    return z.at[:r1, :c1].set(a).at[r1:, c1:].set(b)


@jax.jit
def kernel(cost_features, hardware_features, w_cost, b_cost, g_cost, be_cost,
           w_hw, b_hw, g_hw, be_hw, in_proj_w, in_proj_b, out_proj_w,
           out_proj_b, w_fus, b_fus, g_fus, be_fus, w_out1, b_out1, w_out2,
           b_out2, w_unc1, b_unc1, w_unc2, b_unc2):
    B, CD = cost_features.shape
    grid = (B // (2 * T2),)

    dup = lambda v: jnp.concatenate([v, v]).reshape(1, -1)
    dd = lambda w: _bd(w, w)

    wq = in_proj_w[:H].T; wk = in_proj_w[H:2 * H].T; wv = in_proj_w[2 * H:].T
    bq = in_proj_b[:H]; bk = in_proj_b[H:2 * H]; bv = in_proj_b[2 * H:]
    wqkv = jnp.concatenate([dd(wq), dd(wk), dd(wv)], axis=1)
    bqkv = jnp.concatenate([dup(bq), dup(bk), dup(bv)], axis=1)
    whead = jnp.concatenate([dd(w_out1.T), dd(w_unc1.T)], axis=1)
    bhead = jnp.concatenate([dup(b_out1), dup(b_unc1)], axis=1)

    i = np.arange(2 * H)
    sels = jnp.asarray((i[:, None] // HD == np.arange(16)[None, :])
                       .astype(np.float32) / np.sqrt(HD))
    selt = jnp.asarray((i[None, :] // HD == np.arange(16)[:, None])
                       .astype(np.float32))
    m1 = jnp.asarray(_bd(np.full((H, H), 1.0 / H, np.float32),
                         np.full((H, H), 1.0 / H, np.float32)))

    operands = [
        cost_features, cost_features, hardware_features, hardware_features,
        w_cost.T, dup(b_cost), dup(g_cost), dup(be_cost),
        w_hw.T, dup(b_hw), dup(g_hw), dup(be_hw),
        wqkv, bqkv,
        dd(out_proj_w.T), dup(out_proj_b),
        dd(w_fus.T), dup(b_fus), dup(g_fus), dup(be_fus),
        whead, bhead,
        dd(w_out2.T), dup(b_out2), dd(w_unc2.T), dup(b_unc2),
        m1, sels, selt,
    ]
    full = lambda a: pl.BlockSpec(a.shape, lambda i: (0,) * a.ndim)
    in_specs = [pl.BlockSpec((T2, CD), lambda i: (2 * i, 0)),
                pl.BlockSpec((T2, CD), lambda i: (2 * i + 1, 0)),
                pl.BlockSpec((T2, 8), lambda i: (2 * i, 0)),
                pl.BlockSpec((T2, 8), lambda i: (2 * i + 1, 0))]
    in_specs += [full(a) for a in operands[4:]]

    out_shape = [jax.ShapeDtypeStruct((B, E), jnp.float32),
                 jax.ShapeDtypeStruct((B, E), jnp.float32)]
    out_specs = [pl.BlockSpec((2 * T2, E), lambda i: (i, 0)),
                 pl.BlockSpec((2 * T2, E), lambda i: (i, 0))]

    rb, unc = pl.pallas_call(
        _router_kernel,
        grid=grid,
        in_specs=in_specs,
        out_specs=out_specs,
        out_shape=out_shape,
    )(*operands)
    return rb, unc


# speedup vs baseline: 1.2470x; 1.2470x over previous
import jax
import jax.numpy as jnp
import numpy as np
from jax.experimental import pallas as pl
from jax.experimental.pallas import tpu as pltpu

E = 64
H = 64
NH = 8
HD = H // NH
T2 = 2048


def _gelu(x):
    return 0.5 * x * (1.0 + jax.lax.erf(x * np.float32(1.0 / np.sqrt(2.0))))


def _mmT(x, w):
    return jax.lax.dot_general(x, w, (((1,), (1,)), ((), ())),
                               preferred_element_type=jnp.float32)


def _router_kernel(cfa_ref, cfb_ref, hfa_ref, hfb_ref,
                   wc_ref, bc_ref, gc_ref, bec_ref,
                   wh_ref, bh_ref, gh_ref, beh_ref,
                   wi_ref, bi_ref, wo_ref, bo_ref,
                   wf_ref, bf_ref, gf_ref, bef_ref,
                   w1_ref, b1_ref, w2_ref, b2_ref,
                   wu1_ref, bu1_ref, wu2_ref, bu2_ref,
                   m1_ref, sels_ref, selt_ref,
                   rb_ref, unc_ref,
                   sc_qkv, sc_o, sc_f, sc_head, sc_rb, sc_un):
    f32 = jnp.float32

    @pl.when(pl.program_id(0) == 0)
    def _assemble():
        wi = wi_ref[...]
        sc_qkv[...] = jnp.zeros((3 * 2 * H, 2 * H), f32)
        for j in range(3):
            blk = wi[j * H:(j + 1) * H, :]
            sc_qkv[2 * j * H:(2 * j + 1) * H, 0:H] = blk
            sc_qkv[(2 * j + 1) * H:(2 * j + 2) * H, H:2 * H] = blk
        sc_o[...] = jnp.zeros((2 * H, 2 * H), f32)
        sc_o[0:H, 0:H] = wo_ref[...]
        sc_o[H:2 * H, H:2 * H] = wo_ref[...]
        sc_f[...] = jnp.zeros((2 * H, 2 * H), f32)
        sc_f[0:H, 0:H] = wf_ref[...]
        sc_f[H:2 * H, H:2 * H] = wf_ref[...]
        sc_head[...] = jnp.zeros((96, 2 * H), f32)
        sc_head[0:32, 0:H] = w1_ref[...]
        sc_head[32:64, H:2 * H] = w1_ref[...]
        sc_head[64:80, 0:H] = wu1_ref[...]
        sc_head[80:96, H:2 * H] = wu1_ref[...]
        sc_rb[...] = jnp.zeros((2 * H, H), f32)
        sc_rb[0:H, 0:32] = w2_ref[...]
        sc_rb[H:2 * H, 32:64] = w2_ref[...]
        sc_un[...] = jnp.zeros((2 * H, 32), f32)
        sc_un[0:H, 0:16] = wu2_ref[...]
        sc_un[H:2 * H, 16:32] = wu2_ref[...]

    mm = lambda a, b: jnp.dot(a, b, preferred_element_type=f32)
    m1 = m1_ref[...]
    sels = sels_ref[...]
    selt = selt_ref[...]
    dup = lambda v: jnp.concatenate([v, v], axis=1)

    def segln(x, g, b):
        m = mm(x, m1)
        c = x - m
        v = mm(c * c, m1)
        return c * jax.lax.rsqrt(v + 1e-5) * dup(g) + dup(b)

    wc = wc_ref[...]
    pre_c = jnp.concatenate([_mmT(cfa_ref[...], wc), _mmT(cfb_ref[...], wc)],
                            axis=1) + dup(bc_ref[...])
    ce = _gelu(segln(pre_c, gc_ref[...], bec_ref[...]))
    wh = wh_ref[...]
    pre_h = jnp.concatenate([_mmT(hfa_ref[...], wh), _mmT(hfb_ref[...], wh)],
                            axis=1) + dup(bh_ref[...])
    he = _gelu(segln(pre_h, gh_ref[...], beh_ref[...]))

    bi = bi_ref[...]
    bq = dup(bi[:, 0:H]); bk = dup(bi[:, H:2 * H]); bv = dup(bi[:, 2 * H:])
    bqkv = jnp.concatenate([bq, bk, bv], axis=1)
    qkv_c = _mmT(ce, sc_qkv[...]) + bqkv
    qkv_h = _mmT(he, sc_qkv[...]) + bqkv
    q_c = qkv_c[:, 0:128]; k_c = qkv_c[:, 128:256]; v_c = qkv_c[:, 256:384]
    q_h = qkv_h[:, 0:128]; k_h = qkv_h[:, 128:256]; v_h = qkv_h[:, 256:384]

    s_cc = mm(q_c * k_c, sels)
    s_ch = mm(q_c * k_h, sels)
    s_ct = mm(q_c * bk, sels)
    s_hc = mm(q_h * k_c, sels)
    s_hh = mm(q_h * k_h, sels)
    s_ht = mm(q_h * bk, sels)
    s_tc = mm(bq * k_c, sels)
    s_th = mm(bq * k_h, sels)
    s_tt = mm(bq * bk, sels)

    def softmax3(a, b, c):
        m = jnp.maximum(jnp.maximum(a, b), c)
        ea = jnp.exp(a - m); eb = jnp.exp(b - m); ec = jnp.exp(c - m)
        inv = 1.0 / (ea + eb + ec)
        return ea * inv, eb * inv, ec * inv

    a_cc, a_ch, a_ct = softmax3(s_cc, s_ch, s_ct)
    a_hc, a_hh, a_ht = softmax3(s_hc, s_hh, s_ht)
    a_tc, a_th, a_tt = softmax3(s_tc, s_th, jnp.zeros_like(s_tc) + s_tt)

    third = np.float32(1.0 / 3.0)
    w_vc = (a_cc + a_hc + a_tc) * third
    w_vh = (a_ch + a_hh + a_th) * third
    w_vt = (a_ct + a_ht + a_tt) * third

    o = (mm(w_vc, selt) * v_c + mm(w_vh, selt) * v_h + mm(w_vt, selt) * bv)
    att_mean = _mmT(o, sc_o[...]) + dup(bo_ref[...])

    fused = _gelu(segln(_mmT(att_mean, sc_f[...]) + dup(bf_ref[...]),
                        gf_ref[...], bef_ref[...]))

    bhd = jnp.concatenate([dup(b1_ref[...]), dup(bu1_ref[...])], axis=1)
    hh = _gelu(_mmT(fused, sc_head[...]) + bhd)
    h1 = hh[:, 0:64]
    hu = hh[:, 64:96]
    rb = jnp.tanh(_mmT(h1, sc_rb[...]) + dup(b2_ref[...]))
    unc = jnp.logaddexp(_mmT(hu, sc_un[...]) + dup(bu2_ref[...]), 0.0)

    rb_ref[0:T2, :] = rb[:, 0:E]
    rb_ref[T2:2 * T2, :] = rb[:, E:2 * E]
    unc_ref[0:T2, :] = unc[:, 0:E]
    unc_ref[T2:2 * T2, :] = unc[:, E:2 * E]


@jax.jit
def kernel(cost_features, hardware_features, w_cost, b_cost, g_cost, be_cost,
           w_hw, b_hw, g_hw, be_hw, in_proj_w, in_proj_b, out_proj_w,
           out_proj_b, w_fus, b_fus, g_fus, be_fus, w_out1, b_out1, w_out2,
           b_out2, w_unc1, b_unc1, w_unc2, b_unc2):
    B, CD = cost_features.shape
    grid = (B // (2 * T2),)

    r2 = lambda v: v.reshape(1, -1)

    i = np.arange(2 * H)
    sels = jnp.asarray((i[:, None] // HD == np.arange(16)[None, :])
                       .astype(np.float32) / np.sqrt(HD))
    selt = jnp.asarray((i[None, :] // HD == np.arange(16)[:, None])
                       .astype(np.float32))
    m1z = np.zeros((2 * H, 2 * H), np.float32)
    m1z[:H, :H] = 1.0 / H
    m1z[H:, H:] = 1.0 / H
    m1 = jnp.asarray(m1z)

    operands = [
        cost_features, cost_features, hardware_features, hardware_features,
        w_cost, r2(b_cost), r2(g_cost), r2(be_cost),
        w_hw, r2(b_hw), r2(g_hw), r2(be_hw),
        in_proj_w, r2(in_proj_b), out_proj_w, r2(out_proj_b),
        w_fus, r2(b_fus), r2(g_fus), r2(be_fus),
        w_out1, r2(b_out1), w_out2, r2(b_out2),
        w_unc1, r2(b_unc1), w_unc2, r2(b_unc2),
        m1, sels, selt,
    ]
    full = lambda a: pl.BlockSpec(a.shape, lambda i: (0,) * a.ndim)
    in_specs = [pl.BlockSpec((T2, CD), lambda i: (2 * i, 0)),
                pl.BlockSpec((T2, CD), lambda i: (2 * i + 1, 0)),
                pl.BlockSpec((T2, 8), lambda i: (2 * i, 0)),
                pl.BlockSpec((T2, 8), lambda i: (2 * i + 1, 0))]
    in_specs += [full(a) for a in operands[4:]]

    out_shape = [jax.ShapeDtypeStruct((B, E), jnp.float32),
                 jax.ShapeDtypeStruct((B, E), jnp.float32)]
    out_specs = [pl.BlockSpec((2 * T2, E), lambda i: (i, 0)),
                 pl.BlockSpec((2 * T2, E), lambda i: (i, 0))]

    scratch_shapes = [
        pltpu.VMEM((3 * 2 * H, 2 * H), jnp.float32),
        pltpu.VMEM((2 * H, 2 * H), jnp.float32),
        pltpu.VMEM((2 * H, 2 * H), jnp.float32),
        pltpu.VMEM((96, 2 * H), jnp.float32),
        pltpu.VMEM((2 * H, H), jnp.float32),
        pltpu.VMEM((2 * H, 32), jnp.float32),
    ]

    rb, unc = pl.pallas_call(
        _router_kernel,
        grid=grid,
        in_specs=in_specs,
        out_specs=out_specs,
        out_shape=out_shape,
        scratch_shapes=scratch_shapes,
    )(*operands)
    return rb, unc


# speedup vs baseline: 1.2493x; 1.0018x over previous
import jax
import jax.numpy as jnp
import numpy as np
from jax.experimental import pallas as pl
from jax.experimental.pallas import tpu as pltpu

E = 64
H = 64
NH = 8
HD = H // NH
T2 = 2048


def _gelu(x):
    return 0.5 * x * (1.0 + jax.lax.erf(x * np.float32(1.0 / np.sqrt(2.0))))


def _mmT(x, w):
    return jax.lax.dot_general(x, w, (((1,), (1,)), ((), ())),
                               preferred_element_type=jnp.float32)


def _router_kernel(cfa_ref, cfb_ref, hfa_ref, hfb_ref,
                   wc_ref, bc_ref, gc_ref, bec_ref,
                   wh_ref, bh_ref, gh_ref, beh_ref,
                   wi_ref, bi_ref, wo_ref, bo_ref,
                   wf_ref, bf_ref, gf_ref, bef_ref,
                   w1_ref, b1_ref, w2_ref, b2_ref,
                   wu1_ref, bu1_ref, wu2_ref, bu2_ref,
                   m1_ref, sels_ref, selt_ref,
                   rb_ref, unc_ref,
                   sc_qkv, sc_o, sc_f, sc_head, sc_rb, sc_un):
    f32 = jnp.float32

    @pl.when(pl.program_id(0) == 0)
    def _assemble():
        wi = wi_ref[...]
        sc_qkv[...] = jnp.zeros((3 * 2 * H, 2 * H), f32)
        for j in range(3):
            blk = wi[j * H:(j + 1) * H, :]
            sc_qkv[2 * j * H:(2 * j + 1) * H, 0:H] = blk
            sc_qkv[(2 * j + 1) * H:(2 * j + 2) * H, H:2 * H] = blk
        sc_o[...] = jnp.zeros((2 * H, 2 * H), f32)
        sc_o[0:H, 0:H] = wo_ref[...]
        sc_o[H:2 * H, H:2 * H] = wo_ref[...]
        sc_f[...] = jnp.zeros((2 * H, 2 * H), f32)
        sc_f[0:H, 0:H] = wf_ref[...]
        sc_f[H:2 * H, H:2 * H] = wf_ref[...]
        sc_head[...] = jnp.zeros((96, 2 * H), f32)
        sc_head[0:32, 0:H] = w1_ref[...]
        sc_head[32:64, H:2 * H] = w1_ref[...]
        sc_head[64:80, 0:H] = wu1_ref[...]
        sc_head[80:96, H:2 * H] = wu1_ref[...]
        sc_rb[...] = jnp.zeros((2 * H, H), f32)
        sc_rb[0:H, 0:32] = w2_ref[...]
        sc_rb[H:2 * H, 32:64] = w2_ref[...]
        sc_un[...] = jnp.zeros((2 * H, 32), f32)
        sc_un[0:H, 0:16] = wu2_ref[...]
        sc_un[H:2 * H, 16:32] = wu2_ref[...]

    mm = lambda a, b: jnp.dot(a, b, preferred_element_type=f32)
    m1 = m1_ref[...]
    sels = sels_ref[...]
    selt = selt_ref[...]
    dup = lambda v: jnp.concatenate([v, v], axis=1)

    def segln(x, g, b):
        m = mm(x, m1)
        c = x - m
        v = mm(c * c, m1)
        return c * jax.lax.rsqrt(v + 1e-5) * dup(g) + dup(b)

    wc = wc_ref[...]
    pre_c = jnp.concatenate([_mmT(cfa_ref[...], wc), _mmT(cfb_ref[...], wc)],
                            axis=1) + dup(bc_ref[...].reshape(1, -1))
    ce = _gelu(segln(pre_c, gc_ref[...].reshape(1, -1), bec_ref[...].reshape(1, -1)))
    wh = wh_ref[...]
    pre_h = jnp.concatenate([_mmT(hfa_ref[...], wh), _mmT(hfb_ref[...], wh)],
                            axis=1) + dup(bh_ref[...].reshape(1, -1))
    he = _gelu(segln(pre_h, gh_ref[...].reshape(1, -1), beh_ref[...].reshape(1, -1)))

    bi = bi_ref[...].reshape(1, -1)
    bq = dup(bi[:, 0:H]); bk = dup(bi[:, H:2 * H]); bv = dup(bi[:, 2 * H:])
    bqkv = jnp.concatenate([bq, bk, bv], axis=1)
    qkv_c = _mmT(ce, sc_qkv[...]) + bqkv
    qkv_h = _mmT(he, sc_qkv[...]) + bqkv
    q_c = qkv_c[:, 0:128]; k_c = qkv_c[:, 128:256]; v_c = qkv_c[:, 256:384]
    q_h = qkv_h[:, 0:128]; k_h = qkv_h[:, 128:256]; v_h = qkv_h[:, 256:384]

    s_cc = mm(q_c * k_c, sels)
    s_ch = mm(q_c * k_h, sels)
    s_ct = mm(q_c * bk, sels)
    s_hc = mm(q_h * k_c, sels)
    s_hh = mm(q_h * k_h, sels)
    s_ht = mm(q_h * bk, sels)
    s_tc = mm(bq * k_c, sels)
    s_th = mm(bq * k_h, sels)
    s_tt = mm(bq * bk, sels)

    def softmax3(a, b, c):
        m = jnp.maximum(jnp.maximum(a, b), c)
        ea = jnp.exp(a - m); eb = jnp.exp(b - m); ec = jnp.exp(c - m)
        inv = 1.0 / (ea + eb + ec)
        return ea * inv, eb * inv, ec * inv

    a_cc, a_ch, a_ct = softmax3(s_cc, s_ch, s_ct)
    a_hc, a_hh, a_ht = softmax3(s_hc, s_hh, s_ht)
    a_tc, a_th, a_tt = softmax3(s_tc, s_th, jnp.zeros_like(s_tc) + s_tt)

    third = np.float32(1.0 / 3.0)
    w_vc = (a_cc + a_hc + a_tc) * third
    w_vh = (a_ch + a_hh + a_th) * third
    w_vt = (a_ct + a_ht + a_tt) * third

    o = (mm(w_vc, selt) * v_c + mm(w_vh, selt) * v_h + mm(w_vt, selt) * bv)
    att_mean = _mmT(o, sc_o[...]) + dup(bo_ref[...].reshape(1, -1))

    fused = _gelu(segln(_mmT(att_mean, sc_f[...]) + dup(bf_ref[...].reshape(1, -1)),
                        gf_ref[...].reshape(1, -1), bef_ref[...].reshape(1, -1)))

    bhd = jnp.concatenate([dup(b1_ref[...].reshape(1, -1)), dup(bu1_ref[...].reshape(1, -1))], axis=1)
    hh = _gelu(_mmT(fused, sc_head[...]) + bhd)
    h1 = hh[:, 0:64]
    hu = hh[:, 64:96]
    rb = jnp.tanh(_mmT(h1, sc_rb[...]) + dup(b2_ref[...].reshape(1, -1)))
    unc = jnp.logaddexp(_mmT(hu, sc_un[...]) + dup(bu2_ref[...].reshape(1, -1)), 0.0)

    rb_ref[0:T2, :] = rb[:, 0:E]
    rb_ref[T2:2 * T2, :] = rb[:, E:2 * E]
    unc_ref[0:T2, :] = unc[:, 0:E]
    unc_ref[T2:2 * T2, :] = unc[:, E:2 * E]


@jax.jit
def kernel(cost_features, hardware_features, w_cost, b_cost, g_cost, be_cost,
           w_hw, b_hw, g_hw, be_hw, in_proj_w, in_proj_b, out_proj_w,
           out_proj_b, w_fus, b_fus, g_fus, be_fus, w_out1, b_out1, w_out2,
           b_out2, w_unc1, b_unc1, w_unc2, b_unc2):
    B, CD = cost_features.shape
    grid = (B // (2 * T2),)

    r2 = lambda v: v

    i = np.arange(2 * H)
    sels = jnp.asarray((i[:, None] // HD == np.arange(16)[None, :])
                       .astype(np.float32) / np.sqrt(HD))
    selt = jnp.asarray((i[None, :] // HD == np.arange(16)[:, None])
                       .astype(np.float32))
    m1z = np.zeros((2 * H, 2 * H), np.float32)
    m1z[:H, :H] = 1.0 / H
    m1z[H:, H:] = 1.0 / H
    m1 = jnp.asarray(m1z)

    operands = [
        cost_features, cost_features, hardware_features, hardware_features,
        w_cost, r2(b_cost), r2(g_cost), r2(be_cost),
        w_hw, r2(b_hw), r2(g_hw), r2(be_hw),
        in_proj_w, r2(in_proj_b), out_proj_w, r2(out_proj_b),
        w_fus, r2(b_fus), r2(g_fus), r2(be_fus),
        w_out1, r2(b_out1), w_out2, r2(b_out2),
        w_unc1, r2(b_unc1), w_unc2, r2(b_unc2),
        m1, sels, selt,
    ]
    full = lambda a: pl.BlockSpec(a.shape, lambda i: (0,) * a.ndim)
    in_specs = [pl.BlockSpec((T2, CD), lambda i: (2 * i, 0)),
                pl.BlockSpec((T2, CD), lambda i: (2 * i + 1, 0)),
                pl.BlockSpec((T2, 8), lambda i: (2 * i, 0)),
                pl.BlockSpec((T2, 8), lambda i: (2 * i + 1, 0))]
    in_specs += [full(a) for a in operands[4:]]

    out_shape = [jax.ShapeDtypeStruct((B, E), jnp.float32),
                 jax.ShapeDtypeStruct((B, E), jnp.float32)]
    out_specs = [pl.BlockSpec((2 * T2, E), lambda i: (i, 0)),
                 pl.BlockSpec((2 * T2, E), lambda i: (i, 0))]

    scratch_shapes = [
        pltpu.VMEM((3 * 2 * H, 2 * H), jnp.float32),
        pltpu.VMEM((2 * H, 2 * H), jnp.float32),
        pltpu.VMEM((2 * H, 2 * H), jnp.float32),
        pltpu.VMEM((96, 2 * H), jnp.float32),
        pltpu.VMEM((2 * H, H), jnp.float32),
        pltpu.VMEM((2 * H, 32), jnp.float32),
    ]

    rb, unc = pl.pallas_call(
        _router_kernel,
        grid=grid,
        in_specs=in_specs,
        out_specs=out_specs,
        out_shape=out_shape,
        scratch_shapes=scratch_shapes,
    )(*operands)
    return rb, unc
